# baseline (speedup 1.0000x reference)
import jax
import jax.numpy as jnp
from jax import lax
from jax.experimental import pallas as pl
from jax.experimental.pallas import tpu as pltpu
from jax.experimental.pallas import tpu_sc as plsc

VOCAB = 1000000
DIM = 64
SEQ = 200
BATCH = 4096

NC, NS = 2, 16
NW = NC * NS
SEQ_PER_W = BATCH // NW
G0 = 128
G1 = SEQ - G0
NB = 4


def _sc_kernel(x_hbm, tab_hbm, pos_hbm, out_hbm, pos_v, idx_v, rows_v,
               gsem, osem):
    wid = lax.axis_index("s") * NC + lax.axis_index("c")
    base = wid * SEQ_PER_W

    pltpu.sync_copy(pos_hbm, pos_v)
    pltpu.sync_copy(x_hbm.at[pl.ds(base, SEQ_PER_W)], idx_v)

    def gather(c, b):
        pltpu.async_copy(tab_hbm.at[idx_v.at[c, pl.ds(0, G0)]],
                         rows_v.at[b, pl.ds(0, G0)], gsem.at[b])
        pltpu.async_copy(tab_hbm.at[idx_v.at[c, pl.ds(G0, G1)]],
                         rows_v.at[b, pl.ds(G0, G1)], gsem.at[b])

    def wait_gather(c, b):
        pltpu.make_async_copy(tab_hbm.at[idx_v.at[c, pl.ds(0, G0)]],
                              rows_v.at[b, pl.ds(0, G0)], gsem.at[b]).wait()
        pltpu.make_async_copy(tab_hbm.at[idx_v.at[c, pl.ds(G0, G1)]],
                              rows_v.at[b, pl.ds(G0, G1)], gsem.at[b]).wait()

    gather(0, 0)
    gather(1, 1)

    @pl.loop(0, SEQ_PER_W)
    def chunk_loop(c):
        b = lax.rem(c, NB)

        @pl.when(c + 2 < SEQ_PER_W)
        def _():
            bn = lax.rem(c + 2, NB)

            @pl.when(c >= 2)
            def _():
                pltpu.make_async_copy(rows_v.at[bn],
                                      out_hbm.at[base + c - 2],
                                      osem.at[bn]).wait()

            gather(c + 2, bn)

        wait_gather(c, b)

        @pl.loop(0, SEQ, unroll=8)
        def add_loop(t):
            for d in range(DIM // 16):
                sl = pl.ds(16 * d, 16)
                rows_v[b, t, sl] = rows_v[b, t, sl] + pos_v[t, sl]

        pltpu.async_copy(rows_v.at[b], out_hbm.at[base + c], osem.at[b])

    for k in range(NB):
        c = SEQ_PER_W - NB + k
        pltpu.make_async_copy(rows_v.at[k], out_hbm.at[base + c],
                              osem.at[k]).wait()


@jax.jit
def kernel(x, token_emb, pos_emb):
    xi = x.astype(jnp.int32)
    pos = pos_emb[0, :SEQ, :]

    mesh = plsc.VectorSubcoreMesh(core_axis_name="c", subcore_axis_name="s")
    run = pl.kernel(
        _sc_kernel,
        out_type=jax.ShapeDtypeStruct((BATCH, SEQ, DIM), jnp.float32),
        mesh=mesh,
        scratch_types=[
            pltpu.VMEM((SEQ, DIM), jnp.float32),
            pltpu.VMEM((SEQ_PER_W, SEQ), jnp.int32),
            pltpu.VMEM((NB, SEQ, DIM), jnp.float32),
            pltpu.SemaphoreType.DMA((NB,)),
            pltpu.SemaphoreType.DMA((NB,)),
        ],
        compiler_params=pltpu.CompilerParams(use_tc_tiling_on_sc=False),
    )
    return run(xi, token_emb, pos)
